# Initial kernel scaffold; baseline (speedup 1.0000x reference)
import jax
import jax.numpy as jnp
from jax.experimental import pallas as pl


def kernel(x_card, x_user, edge_index_user_card, edge_index_card_user, W_in_card, b_in_card, W_in_user, b_in_user, Wl_u2c_0, bl_u2c_0, Wr_u2c_0, Wl_c2u_0, bl_c2u_0, Wr_c2u_0, Wl_u2c_1, bl_u2c_1, Wr_u2c_1, Wl_c2u_1, bl_c2u_1, Wr_c2u_1, W_cls, b_cls):
    raise NotImplementedError("write your pallas kernel here")


# speedup vs baseline: 1.0332x; 1.0332x over previous
import functools

import jax
import jax.numpy as jnp
from jax import lax
from jax.experimental import pallas as pl
from jax.experimental.pallas import tpu as pltpu
from jax.experimental.pallas import tpu_sc as plsc

N = 10000
E = 160000
H = 512
NC, NS = 2, 16
NW = NC * NS
EB = 128
PW = 5120
E_PAD = NW * PW
CH = 128
NP = 10112
RPT = NP // NS
CW = 16

_sc_mesh = plsc.VectorSubcoreMesh(
    core_axis_name="c", subcore_axis_name="s", num_cores=NC, num_subcores=NS)


@functools.partial(
    pl.kernel,
    out_type=jax.ShapeDtypeStruct((NC, NP, CH), jnp.float32),
    mesh=_sc_mesh,
    scratch_types=[
        pltpu.VMEM((EB,), jnp.int32),
        pltpu.VMEM((EB,), jnp.int32),
        pltpu.VMEM((EB, CH), jnp.float32),
        pltpu.SemaphoreType.DMA,
        pltpu.VMEM_SHARED((NP, CH), jnp.float32),
    ],
)
def _segsum_sc(table, src, dst, zeros, out, idxs_v, idxd_v, rows_v, sem, accum):
    cid = lax.axis_index("c")
    sid = lax.axis_index("s")
    wid = cid * NS + sid

    pltpu.sync_copy(zeros.at[pl.ds(sid * RPT, RPT)],
                    accum.at[pl.ds(sid * RPT, RPT)])
    plsc.subcore_barrier()

    def body(i, carry):
        base = wid * PW + i * EB
        pltpu.sync_copy(src.at[pl.ds(base, EB)], idxs_v)
        pltpu.sync_copy(dst.at[pl.ds(base, EB)], idxd_v)
        pltpu.async_copy(table.at[idxs_v], rows_v, sem).wait()
        pltpu.sync_copy(rows_v, accum.at[idxd_v], add=True)
        return carry

    lax.fori_loop(0, PW // EB, body, 0)
    plsc.subcore_barrier()

    pltpu.sync_copy(accum.at[pl.ds(sid * RPT, RPT)],
                    out.at[cid, pl.ds(sid * RPT, RPT)])


BM = 1000


_PREC = lax.Precision.HIGHEST


def _proj_body(x_ref, w_ref, b_ref, o_ref):
    acc = jnp.dot(x_ref[...], w_ref[...], precision=_PREC,
                  preferred_element_type=jnp.float32)
    o_ref[...] = jnp.maximum(acc + b_ref[...], 0.0)


def _proj(x, w, b):
    d = x.shape[1]
    return pl.pallas_call(
        _proj_body,
        grid=(N // BM,),
        in_specs=[
            pl.BlockSpec((BM, d), lambda i: (i, 0)),
            pl.BlockSpec((d, H), lambda i: (0, 0)),
            pl.BlockSpec((1, H), lambda i: (0, 0)),
        ],
        out_specs=pl.BlockSpec((BM, H), lambda i: (i, 0)),
        out_shape=jax.ShapeDtypeStruct((N, H), jnp.float32),
    )(x, w, b.reshape(1, H))


def _sage_body(relu, p0, p1, p2, p3, cnt, x_ref, wl, bl, wr, o_ref):
    c = cnt[0] + cnt[1]
    inv = 1.0 / jnp.maximum(c[:, :1], 1.0)
    acc = jnp.dot(x_ref[...], wr[...], precision=_PREC,
                  preferred_element_type=jnp.float32)
    for k, p in enumerate((p0, p1, p2, p3)):
        m = (p[0] + p[1]) * inv
        acc += jnp.dot(m, wl[k * CH:(k + 1) * CH, :], precision=_PREC,
                       preferred_element_type=jnp.float32)
    acc += bl[...]
    o_ref[...] = jnp.maximum(acc, 0.0) if relu else acc


def _sage_cls_body(p0, p1, p2, p3, cnt, x_ref, wl, bl, wr, wc, bc,
                   o_ref, lg_ref):
    c = cnt[0] + cnt[1]
    inv = 1.0 / jnp.maximum(c[:, :1], 1.0)
    acc = jnp.dot(x_ref[...], wr[...], precision=_PREC,
                  preferred_element_type=jnp.float32)
    for k, p in enumerate((p0, p1, p2, p3)):
        m = (p[0] + p[1]) * inv
        acc += jnp.dot(m, wl[k * CH:(k + 1) * CH, :], precision=_PREC,
                       preferred_element_type=jnp.float32)
    acc += bl[...]
    o_ref[...] = acc
    lg_ref[...] = jnp.dot(acc, wc[...], precision=_PREC,
                          preferred_element_type=jnp.float32) + bc[...]


_P_SPEC = pl.BlockSpec((NC, BM, CH), lambda i: (0, i, 0))
_CNT_SPEC = pl.BlockSpec((NC, BM, CH), lambda i: (0, i, 0))
_X_SPEC = pl.BlockSpec((BM, H), lambda i: (i, 0))
_WL_SPEC = pl.BlockSpec((H, H), lambda i: (0, 0))
_B_SPEC = pl.BlockSpec((1, H), lambda i: (0, 0))


def _sage_layer(parts, cnt, x, wl, bl, wr, relu):
    return pl.pallas_call(
        functools.partial(_sage_body, relu),
        grid=(N // BM,),
        in_specs=[_P_SPEC] * 4 + [_CNT_SPEC, _X_SPEC, _WL_SPEC, _B_SPEC,
                                  _WL_SPEC],
        out_specs=_X_SPEC,
        out_shape=jax.ShapeDtypeStruct((N, H), jnp.float32),
    )(*parts, cnt, x, wl, bl.reshape(1, H), wr)


def _sage_layer_cls(parts, cnt, x, wl, bl, wr, wc, bc):
    return pl.pallas_call(
        _sage_cls_body,
        grid=(N // BM,),
        in_specs=[_P_SPEC] * 4 + [
            _CNT_SPEC, _X_SPEC, _WL_SPEC, _B_SPEC, _WL_SPEC,
            pl.BlockSpec((H, 128), lambda i: (0, 0)),
            pl.BlockSpec((1, 128), lambda i: (0, 0)),
        ],
        out_specs=[_X_SPEC, pl.BlockSpec((BM, 128), lambda i: (i, 0))],
        out_shape=[jax.ShapeDtypeStruct((N, H), jnp.float32),
                   jax.ShapeDtypeStruct((N, 128), jnp.float32)],
    )(*parts, cnt, x, wl, bl.reshape(1, H), wr, wc, bc)


def _segmean_parts(table, src, dst, zeros):
    return [_segsum_sc(table[:, k * CH:(k + 1) * CH], src, dst, zeros)
            for k in range(H // CH)]


def kernel(x_card, x_user, edge_index_user_card, edge_index_card_user,
           W_in_card, b_in_card, W_in_user, b_in_user,
           Wl_u2c_0, bl_u2c_0, Wr_u2c_0, Wl_c2u_0, bl_c2u_0, Wr_c2u_0,
           Wl_u2c_1, bl_u2c_1, Wr_u2c_1, Wl_c2u_1, bl_c2u_1, Wr_c2u_1,
           W_cls, b_cls):
    pad0 = jnp.zeros((E_PAD - E,), jnp.int32)
    padN = jnp.full((E_PAD - E,), N, jnp.int32)
    src_uc = jnp.concatenate([edge_index_user_card[0], pad0])
    dst_uc = jnp.concatenate([edge_index_user_card[1], padN])
    src_cu = jnp.concatenate([edge_index_card_user[0], pad0])
    dst_cu = jnp.concatenate([edge_index_card_user[1], padN])

    zeros = jnp.zeros((NP, CH), jnp.float32)
    ones_table = jnp.ones((N, CH), jnp.float32)

    cnt_c = _segsum_sc(ones_table, src_uc, dst_uc, zeros)
    cnt_u = _segsum_sc(ones_table, src_cu, dst_cu, zeros)

    h_card = _proj(x_card, W_in_card, b_in_card)
    h_user = _proj(x_user, W_in_user, b_in_user)

    parts_c = _segmean_parts(h_user, src_uc, dst_uc, zeros)
    parts_u = _segmean_parts(h_card, src_cu, dst_cu, zeros)
    h_card1 = _sage_layer(parts_c, cnt_c, h_card, Wl_u2c_0, bl_u2c_0,
                          Wr_u2c_0, relu=True)
    h_user1 = _sage_layer(parts_u, cnt_u, h_user, Wl_c2u_0, bl_c2u_0,
                          Wr_c2u_0, relu=True)

    parts_c = _segmean_parts(h_user1, src_uc, dst_uc, zeros)
    parts_u = _segmean_parts(h_card1, src_cu, dst_cu, zeros)
    wc_pad = jnp.zeros((H, 128), jnp.float32).at[:, :2].set(W_cls)
    bc_pad = jnp.zeros((1, 128), jnp.float32).at[0, :2].set(b_cls)
    h_card2, logits_pad = _sage_layer_cls(parts_c, cnt_c, h_card1, Wl_u2c_1,
                                          bl_u2c_1, Wr_u2c_1, wc_pad, bc_pad)
    h_user2 = _sage_layer(parts_u, cnt_u, h_user1, Wl_c2u_1, bl_c2u_1,
                          Wr_c2u_1, relu=False)

    return logits_pad[:, :2], h_card2, h_user2


# speedup vs baseline: 1.2985x; 1.2568x over previous
import functools

import jax
import jax.numpy as jnp
from jax import lax
from jax.experimental import pallas as pl
from jax.experimental.pallas import tpu as pltpu
from jax.experimental.pallas import tpu_sc as plsc

N = 10000
E = 160000
H = 512
NC, NS = 2, 16
NW = NC * NS
EB = 128
NB = 40
PW = NB * EB
E_PAD = NW * PW
CH = 128
NK = H // CH
NP = 10240
RPT = NP // NS
NBUF = 2

_sc_mesh = plsc.VectorSubcoreMesh(
    core_axis_name="c", subcore_axis_name="s", num_cores=NC, num_subcores=NS)


def _conv_body(*refs):
    tables = refs[:NK]
    ones, src2d, dst2d, zeros, out = refs[NK:NK + 5]
    srcs_v, dsts_v = refs[NK + 5:NK + 7]
    rows = refs[NK + 7:NK + 7 + NBUF]
    sems = refs[NK + 7 + NBUF:NK + 7 + 2 * NBUF]
    accum = refs[NK + 7 + 2 * NBUF]

    cid = lax.axis_index("c")
    sid = lax.axis_index("s")
    wid = cid * NS + sid
    stripe = pl.ds(sid * RPT, RPT)

    pltpu.sync_copy(src2d.at[pl.ds(wid * NB, NB)], srcs_v)
    pltpu.sync_copy(dst2d.at[pl.ds(wid * NB, NB)], dsts_v)

    def _zero_stripe():
        for t in range(RPT // EB):
            pltpu.sync_copy(zeros,
                            accum.at[pl.ds(sid * RPT + t * EB, EB)])

    for c in range(NK):
        table = tables[c]
        _zero_stripe()
        plsc.subcore_barrier()

        for b in range(NBUF):
            pltpu.async_copy(table.at[srcs_v.at[b]], rows[b], sems[b])

        def group(g, carry):
            for b in range(NBUF):
                j = g * NBUF + b
                pltpu.make_async_copy(table.at[pl.ds(0, EB)],
                                      rows[b], sems[b]).wait()
                pltpu.sync_copy(rows[b], accum.at[dsts_v.at[j]], add=True)

                @pl.when(j + NBUF < NB)
                def _prefetch(b=b, j=j, table=table):
                    pltpu.async_copy(table.at[srcs_v.at[j + NBUF]],
                                     rows[b], sems[b])
            return carry

        lax.fori_loop(0, NB // NBUF, group, 0)
        plsc.subcore_barrier()
        pltpu.sync_copy(accum.at[stripe], out.at[c, cid, stripe])

    _zero_stripe()
    plsc.subcore_barrier()
    pltpu.sync_copy(ones, rows[0])

    def cgroup(j, carry):
        pltpu.sync_copy(rows[0], accum.at[dsts_v.at[j]], add=True)
        return carry

    lax.fori_loop(0, NB, cgroup, 0)
    plsc.subcore_barrier()
    pltpu.sync_copy(accum.at[stripe], out.at[NK, cid, stripe])


_conv_sc = pl.kernel(
    _conv_body,
    out_type=jax.ShapeDtypeStruct((NK + 1, NC, NP, CH), jnp.float32),
    mesh=_sc_mesh,
    scratch_types=(
        [pltpu.VMEM((NB, EB), jnp.int32)] * 2
        + [pltpu.VMEM((EB, CH), jnp.float32)] * NBUF
        + [pltpu.SemaphoreType.DMA] * NBUF
        + [pltpu.VMEM_SHARED((NP, CH), jnp.float32)]
    ),
)


BM = 1000
_PREC = lax.Precision.HIGHEST


def _proj_body(x_ref, w_ref, b_ref, o_ref):
    acc = jnp.dot(x_ref[...], w_ref[...], precision=_PREC,
                  preferred_element_type=jnp.float32)
    o_ref[...] = jnp.maximum(acc + b_ref[...], 0.0)


def _proj(x, w, b):
    d = x.shape[1]
    return pl.pallas_call(
        _proj_body,
        grid=(N // BM,),
        in_specs=[
            pl.BlockSpec((BM, d), lambda i: (i, 0)),
            pl.BlockSpec((d, H), lambda i: (0, 0)),
            pl.BlockSpec((1, H), lambda i: (0, 0)),
        ],
        out_specs=pl.BlockSpec((BM, H), lambda i: (i, 0)),
        out_shape=jax.ShapeDtypeStruct((N, H), jnp.float32),
    )(x, w, b.reshape(1, H))


def _mean_matmul(parts, cnt, x_ref, wl, wr, bl):
    c = cnt[0, 0] + cnt[0, 1]
    inv = 1.0 / jnp.maximum(c[:, :1], 1.0)
    acc = jnp.dot(x_ref[...], wr[...], precision=_PREC,
                  preferred_element_type=jnp.float32)
    for k in range(NK):
        m = (parts[k, 0] + parts[k, 1]) * inv
        acc += jnp.dot(m, wl[k * CH:(k + 1) * CH, :], precision=_PREC,
                       preferred_element_type=jnp.float32)
    return acc + bl[...]


def _sage_body(relu, p_ref, cnt_ref, x_ref, wl, bl, wr, o_ref):
    acc = _mean_matmul(p_ref, cnt_ref, x_ref, wl, wr, bl)
    o_ref[...] = jnp.maximum(acc, 0.0) if relu else acc


def _sage_cls_body(p_ref, cnt_ref, x_ref, wl, bl, wr, wc, bc, o_ref, lg_ref):
    acc = _mean_matmul(p_ref, cnt_ref, x_ref, wl, wr, bl)
    o_ref[...] = acc
    lg_ref[...] = jnp.dot(acc, wc[...], precision=_PREC,
                          preferred_element_type=jnp.float32) + bc[...]


_P_SPEC = pl.BlockSpec((NK, NC, BM, CH), lambda i: (0, 0, i, 0))
_CNT_SPEC = pl.BlockSpec((1, NC, BM, CH), lambda i: (NK, 0, i, 0))
_X_SPEC = pl.BlockSpec((BM, H), lambda i: (i, 0))
_WL_SPEC = pl.BlockSpec((H, H), lambda i: (0, 0))
_B_SPEC = pl.BlockSpec((1, H), lambda i: (0, 0))


def _sage_layer(parts, cnt, x, wl, bl, wr, relu):
    return pl.pallas_call(
        functools.partial(_sage_body, relu),
        grid=(N // BM,),
        in_specs=[_P_SPEC, _CNT_SPEC, _X_SPEC, _WL_SPEC, _B_SPEC, _WL_SPEC],
        out_specs=_X_SPEC,
        out_shape=jax.ShapeDtypeStruct((N, H), jnp.float32),
    )(parts, cnt, x, wl, bl.reshape(1, H), wr)


def _sage_layer_cls(parts, cnt, x, wl, bl, wr, wc, bc):
    return pl.pallas_call(
        _sage_cls_body,
        grid=(N // BM,),
        in_specs=[
            _P_SPEC, _CNT_SPEC, _X_SPEC, _WL_SPEC, _B_SPEC, _WL_SPEC,
            pl.BlockSpec((H, 128), lambda i: (0, 0)),
            pl.BlockSpec((1, 128), lambda i: (0, 0)),
        ],
        out_specs=[_X_SPEC, pl.BlockSpec((BM, 128), lambda i: (i, 0))],
        out_shape=[jax.ShapeDtypeStruct((N, H), jnp.float32),
                   jax.ShapeDtypeStruct((N, 128), jnp.float32)],
    )(parts, cnt, x, wl, bl.reshape(1, H), wr, wc, bc)


def _chunks(h):
    return [h[:, k * CH:(k + 1) * CH] for k in range(NK)]


def kernel(x_card, x_user, edge_index_user_card, edge_index_card_user,
           W_in_card, b_in_card, W_in_user, b_in_user,
           Wl_u2c_0, bl_u2c_0, Wr_u2c_0, Wl_c2u_0, bl_c2u_0, Wr_c2u_0,
           Wl_u2c_1, bl_u2c_1, Wr_u2c_1, Wl_c2u_1, bl_c2u_1, Wr_c2u_1,
           W_cls, b_cls):
    pad0 = jnp.zeros((E_PAD - E,), jnp.int32)
    padN = jnp.full((E_PAD - E,), N, jnp.int32)
    src_uc = jnp.concatenate([edge_index_user_card[0], pad0]).reshape(-1, EB)
    dst_uc = jnp.concatenate([edge_index_user_card[1], padN]).reshape(-1, EB)
    src_cu = jnp.concatenate([edge_index_card_user[0], pad0]).reshape(-1, EB)
    dst_cu = jnp.concatenate([edge_index_card_user[1], padN]).reshape(-1, EB)

    zeros = jnp.zeros((EB, CH), jnp.float32)
    ones = jnp.ones((EB, CH), jnp.float32)

    h_card = _proj(x_card, W_in_card, b_in_card)
    h_user = _proj(x_user, W_in_user, b_in_user)

    pc0 = _conv_sc(*_chunks(h_user), ones, src_uc, dst_uc, zeros)
    pu0 = _conv_sc(*_chunks(h_card), ones, src_cu, dst_cu, zeros)
    h_card1 = _sage_layer(pc0, pc0, h_card, Wl_u2c_0, bl_u2c_0,
                          Wr_u2c_0, relu=True)
    h_user1 = _sage_layer(pu0, pu0, h_user, Wl_c2u_0, bl_c2u_0,
                          Wr_c2u_0, relu=True)

    pc1 = _conv_sc(*_chunks(h_user1), ones, src_uc, dst_uc, zeros)
    pu1 = _conv_sc(*_chunks(h_card1), ones, src_cu, dst_cu, zeros)
    wc_pad = jnp.zeros((H, 128), jnp.float32).at[:, :2].set(W_cls)
    bc_pad = jnp.zeros((1, 128), jnp.float32).at[0, :2].set(b_cls)
    h_card2, logits_pad = _sage_layer_cls(pc1, pc0, h_card1, Wl_u2c_1,
                                          bl_u2c_1, Wr_u2c_1, wc_pad, bc_pad)
    h_user2 = _sage_layer(pu1, pu0, h_user1, Wl_c2u_1, bl_c2u_1,
                          Wr_c2u_1, relu=False)

    return logits_pad[:, :2], h_card2, h_user2


# speedup vs baseline: 1.3579x; 1.0457x over previous
import functools

import jax
import jax.numpy as jnp
from jax import lax
from jax.experimental import pallas as pl
from jax.experimental.pallas import tpu as pltpu
from jax.experimental.pallas import tpu_sc as plsc

N = 10000
E = 160000
H = 512
NC, NS = 2, 16
NW = NC * NS
EB = 128
NB = 40
PW = NB * EB
E_PAD = NW * PW
CH = 128
NK = H // CH
NP = 10240
RPT = NP // NS
NBUF = 2

_sc_mesh = plsc.VectorSubcoreMesh(
    core_axis_name="c", subcore_axis_name="s", num_cores=NC, num_subcores=NS)


def _conv_body(with_counts, *refs):
    tables = refs[:NK]
    ones, src2d, dst2d, zeros, out = refs[NK:NK + 5]
    srcs_v, dsts_v = refs[NK + 5:NK + 7]
    rows = refs[NK + 7:NK + 7 + NBUF]
    sems = refs[NK + 7 + NBUF:NK + 7 + 2 * NBUF]
    accum = refs[NK + 7 + 2 * NBUF]

    cid = lax.axis_index("c")
    sid = lax.axis_index("s")
    wid = cid * NS + sid
    stripe = pl.ds(sid * RPT, RPT)

    pltpu.sync_copy(src2d.at[pl.ds(wid * NB, NB)], srcs_v)
    pltpu.sync_copy(dst2d.at[pl.ds(wid * NB, NB)], dsts_v)

    def _zero_stripe():
        for t in range(RPT // EB):
            pltpu.sync_copy(zeros,
                            accum.at[pl.ds(sid * RPT + t * EB, EB)])

    for c in range(NK):
        table = tables[c]
        _zero_stripe()
        plsc.subcore_barrier()

        for b in range(NBUF):
            pltpu.async_copy(table.at[srcs_v.at[b]], rows[b], sems[b])

        def group(g, carry):
            for b in range(NBUF):
                j = g * NBUF + b
                pltpu.make_async_copy(table.at[pl.ds(0, EB)],
                                      rows[b], sems[b]).wait()
                pltpu.sync_copy(rows[b], accum.at[dsts_v.at[j]], add=True)

                @pl.when(j + NBUF < NB)
                def _prefetch(b=b, j=j, table=table):
                    pltpu.async_copy(table.at[srcs_v.at[j + NBUF]],
                                     rows[b], sems[b])
            return carry

        lax.fori_loop(0, NB // NBUF, group, 0)
        plsc.subcore_barrier()
        pltpu.sync_copy(accum.at[stripe], out.at[c, cid, stripe])

    if with_counts:
        _zero_stripe()
        plsc.subcore_barrier()
        pltpu.sync_copy(ones, rows[0])

        def cgroup(j, carry):
            pltpu.sync_copy(rows[0], accum.at[dsts_v.at[j]], add=True)
            return carry

        lax.fori_loop(0, NB, cgroup, 0)
        plsc.subcore_barrier()
        pltpu.sync_copy(accum.at[stripe], out.at[NK, cid, stripe])


def _make_conv(with_counts):
    return pl.kernel(
        functools.partial(_conv_body, with_counts),
        out_type=jax.ShapeDtypeStruct(
            (NK + (1 if with_counts else 0), NC, NP, CH), jnp.float32),
        mesh=_sc_mesh,
        scratch_types=(
            [pltpu.VMEM((NB, EB), jnp.int32)] * 2
            + [pltpu.VMEM((EB, CH), jnp.float32)] * NBUF
            + [pltpu.SemaphoreType.DMA] * NBUF
            + [pltpu.VMEM_SHARED((NP, CH), jnp.float32)]
        ),
    )


_conv_sc = _make_conv(True)
_conv_sc_nc = _make_conv(False)


BM = 1000
_PREC = lax.Precision.HIGHEST


def _proj_body(x_ref, w_ref, b_ref, o_ref):
    acc = jnp.dot(x_ref[...], w_ref[...], precision=_PREC,
                  preferred_element_type=jnp.float32)
    o_ref[...] = jnp.maximum(acc + b_ref[...], 0.0)


def _proj(x, w, b):
    d = x.shape[1]
    return pl.pallas_call(
        _proj_body,
        grid=(N // BM,),
        in_specs=[
            pl.BlockSpec((BM, d), lambda i: (i, 0)),
            pl.BlockSpec((d, H), lambda i: (0, 0)),
            pl.BlockSpec((1, H), lambda i: (0, 0)),
        ],
        out_specs=pl.BlockSpec((BM, H), lambda i: (i, 0)),
        out_shape=jax.ShapeDtypeStruct((N, H), jnp.float32),
    )(x, w, b.reshape(1, H))


def _mean_matmul(parts, cnt, x_ref, wl, wr, bl):
    c = cnt[0, 0] + cnt[0, 1]
    inv = 1.0 / jnp.maximum(c[:, :1], 1.0)
    acc = jnp.dot(x_ref[...], wr[...], precision=_PREC,
                  preferred_element_type=jnp.float32)
    for k in range(NK):
        m = (parts[k, 0] + parts[k, 1]) * inv
        acc += jnp.dot(m, wl[k * CH:(k + 1) * CH, :], precision=_PREC,
                       preferred_element_type=jnp.float32)
    return acc + bl[...]


def _sage_body(relu, p_ref, cnt_ref, x_ref, wl, bl, wr, o_ref):
    acc = _mean_matmul(p_ref, cnt_ref, x_ref, wl, wr, bl)
    o_ref[...] = jnp.maximum(acc, 0.0) if relu else acc


def _sage_cls_body(p_ref, cnt_ref, x_ref, wl, bl, wr, wc, bc, o_ref, lg_ref):
    acc = _mean_matmul(p_ref, cnt_ref, x_ref, wl, wr, bl)
    o_ref[...] = acc
    lg_ref[...] = jnp.dot(acc, wc[...], precision=_PREC,
                          preferred_element_type=jnp.float32) + bc[...]


_P_SPEC = pl.BlockSpec((NK, NC, BM, CH), lambda i: (0, 0, i, 0))
_CNT_SPEC = pl.BlockSpec((1, NC, BM, CH), lambda i: (NK, 0, i, 0))
_X_SPEC = pl.BlockSpec((BM, H), lambda i: (i, 0))
_WL_SPEC = pl.BlockSpec((H, H), lambda i: (0, 0))
_B_SPEC = pl.BlockSpec((1, H), lambda i: (0, 0))


def _sage_layer(parts, cnt, x, wl, bl, wr, relu):
    return pl.pallas_call(
        functools.partial(_sage_body, relu),
        grid=(N // BM,),
        in_specs=[_P_SPEC, _CNT_SPEC, _X_SPEC, _WL_SPEC, _B_SPEC, _WL_SPEC],
        out_specs=_X_SPEC,
        out_shape=jax.ShapeDtypeStruct((N, H), jnp.float32),
    )(parts, cnt, x, wl, bl.reshape(1, H), wr)


def _sage_layer_cls(parts, cnt, x, wl, bl, wr, wc, bc):
    return pl.pallas_call(
        _sage_cls_body,
        grid=(N // BM,),
        in_specs=[
            _P_SPEC, _CNT_SPEC, _X_SPEC, _WL_SPEC, _B_SPEC, _WL_SPEC,
            pl.BlockSpec((H, 128), lambda i: (0, 0)),
            pl.BlockSpec((1, 128), lambda i: (0, 0)),
        ],
        out_specs=[_X_SPEC, pl.BlockSpec((BM, 128), lambda i: (i, 0))],
        out_shape=[jax.ShapeDtypeStruct((N, H), jnp.float32),
                   jax.ShapeDtypeStruct((N, 128), jnp.float32)],
    )(parts, cnt, x, wl, bl.reshape(1, H), wr, wc, bc)


def _chunks(h):
    return [h[:, k * CH:(k + 1) * CH] for k in range(NK)]


def kernel(x_card, x_user, edge_index_user_card, edge_index_card_user,
           W_in_card, b_in_card, W_in_user, b_in_user,
           Wl_u2c_0, bl_u2c_0, Wr_u2c_0, Wl_c2u_0, bl_c2u_0, Wr_c2u_0,
           Wl_u2c_1, bl_u2c_1, Wr_u2c_1, Wl_c2u_1, bl_c2u_1, Wr_c2u_1,
           W_cls, b_cls):
    pad0 = jnp.zeros((E_PAD - E,), jnp.int32)
    padN = jnp.full((E_PAD - E,), N, jnp.int32)
    src_uc = jnp.concatenate([edge_index_user_card[0], pad0]).reshape(-1, EB)
    dst_uc = jnp.concatenate([edge_index_user_card[1], padN]).reshape(-1, EB)
    src_cu = jnp.concatenate([edge_index_card_user[0], pad0]).reshape(-1, EB)
    dst_cu = jnp.concatenate([edge_index_card_user[1], padN]).reshape(-1, EB)

    zeros = jnp.zeros((EB, CH), jnp.float32)
    ones = jnp.ones((EB, CH), jnp.float32)

    h_card = _proj(x_card, W_in_card, b_in_card)
    h_user = _proj(x_user, W_in_user, b_in_user)

    pc0 = _conv_sc(*_chunks(h_user), ones, src_uc, dst_uc, zeros)
    pu0 = _conv_sc(*_chunks(h_card), ones, src_cu, dst_cu, zeros)
    h_card1 = _sage_layer(pc0, pc0, h_card, Wl_u2c_0, bl_u2c_0,
                          Wr_u2c_0, relu=True)
    h_user1 = _sage_layer(pu0, pu0, h_user, Wl_c2u_0, bl_c2u_0,
                          Wr_c2u_0, relu=True)

    pc1 = _conv_sc_nc(*_chunks(h_user1), ones, src_uc, dst_uc, zeros)
    pu1 = _conv_sc_nc(*_chunks(h_card1), ones, src_cu, dst_cu, zeros)
    wc_pad = jnp.zeros((H, 128), jnp.float32).at[:, :2].set(W_cls)
    bc_pad = jnp.zeros((1, 128), jnp.float32).at[0, :2].set(b_cls)
    h_card2, logits_pad = _sage_layer_cls(pc1, pc0, h_card1, Wl_u2c_1,
                                          bl_u2c_1, Wr_u2c_1, wc_pad, bc_pad)
    h_user2 = _sage_layer(pu1, pu0, h_user1, Wl_c2u_1, bl_c2u_1,
                          Wr_c2u_1, relu=False)

    return logits_pad[:, :2], h_card2, h_user2
